# baseline (speedup 1.0000x reference)
import functools

import jax
import jax.numpy as jnp
from jax import lax
from jax.experimental import pallas as pl
from jax.experimental.pallas import tpu as pltpu
from jax.experimental.pallas import tpu_sc as plsc

_NC, _NS, _LANES = 2, 16, 16


def _gc_kernel(freqs_ref, gc_ref):
    f = freqs_ref[...]
    g = f.shape[0]
    ones = jnp.ones((1, g), jnp.float32)
    gcx = lax.dot_general(ones, f, (((1,), (1,)), ((), ())),
                          preferred_element_type=jnp.float32)
    gcy = lax.dot_general(ones, f, (((1,), (0,)), ((), ())),
                          preferred_element_type=jnp.float32)
    gc_ref[...] = jnp.concatenate([gcx, gcy], axis=0)


def _compute_gc(freqs):
    g = freqs.shape[0]
    return pl.pallas_call(
        _gc_kernel,
        out_shape=jax.ShapeDtypeStruct((2, g), jnp.float32),
    )(freqs)


def _sc_mesh():
    return plsc.VectorSubcoreMesh(core_axis_name="c", subcore_axis_name="s",
                                  num_cores=_NC, num_subcores=_NS)


def _sc_gather(idx0, idx1, gc):
    b, l = idx0.shape
    g = gc.shape[1]
    nw = _NC * _NS
    bpw = b // nw
    chunks = l // _LANES

    @functools.partial(
        pl.kernel,
        out_type=jax.ShapeDtypeStruct((b, l), jnp.float32),
        mesh=_sc_mesh(),
        compiler_params=pltpu.CompilerParams(needs_layout_passes=False),
        scratch_types=[
            pltpu.VMEM((g,), jnp.float32),
            pltpu.VMEM((g,), jnp.float32),
            pltpu.VMEM((bpw, l), jnp.int32),
            pltpu.VMEM((bpw, l), jnp.int32),
            pltpu.VMEM((bpw, l), jnp.float32),
            pltpu.SemaphoreType.DMA,
            pltpu.SemaphoreType.DMA,
        ],
    )
    def run(idx0_hbm, idx1_hbm, gc_hbm, out_hbm, gcx_v, gcy_v, i0_v, i1_v,
            f_v, sem_in, sem_out):
        wid = lax.axis_index("s") * _NC + lax.axis_index("c")
        base = wid * bpw
        cps = [pltpu.async_copy(gc_hbm.at[0], gcx_v, sem_in),
               pltpu.async_copy(gc_hbm.at[1], gcy_v, sem_in)]
        for j in range(bpw):
            cps.append(pltpu.async_copy(idx0_hbm.at[base + j], i0_v.at[j], sem_in))
            cps.append(pltpu.async_copy(idx1_hbm.at[base + j], i1_v.at[j], sem_in))
        for c in cps:
            c.wait()
        out_cps = []
        for j in range(bpw):
            def body(c, carry, j=j):
                off = c * _LANES
                v0 = i0_v[j, pl.ds(off, _LANES)]
                v1 = i1_v[j, pl.ds(off, _LANES)]
                fx = plsc.load_gather(gcx_v, [v0])
                fy = plsc.load_gather(gcy_v, [v1])
                f_v[j, pl.ds(off, _LANES)] = fx + fy
                return carry

            lax.fori_loop(0, chunks, body, 0, unroll=4)
            out_cps.append(pltpu.async_copy(f_v.at[j], out_hbm.at[base + j],
                                            sem_out))
        for c in out_cps:
            c.wait()

    return run(idx0, idx1, gc)


_TCH = 64


def _sc_pool(idx0, idx1, gc, lengths, feat, base):
    b_total, l = idx0.shape
    _, _, d = feat.shape
    g = gc.shape[1]
    nw = _NC * _NS
    nb = nw
    nt = l // _TCH
    dch = d // _LANES
    wchunks = l // _LANES

    @functools.partial(
        pl.kernel,
        out_type=jax.ShapeDtypeStruct((nb, d), jnp.float32),
        mesh=_sc_mesh(),
        compiler_params=pltpu.CompilerParams(needs_layout_passes=False),
        scratch_types=[
            pltpu.VMEM((g,), jnp.float32),
            pltpu.VMEM((g,), jnp.float32),
            pltpu.VMEM((l,), jnp.int32),
            pltpu.VMEM((l,), jnp.int32),
            pltpu.VMEM((l,), jnp.float32),
            pltpu.VMEM((b_total,), jnp.int32),
            pltpu.VMEM((2, _TCH, d), jnp.float32),
            pltpu.VMEM((d,), jnp.float32),
            pltpu.SemaphoreType.DMA,
            pltpu.SemaphoreType.DMA,
            pltpu.SemaphoreType.DMA,
        ],
    )
    def run(idx0_hbm, idx1_hbm, gc_hbm, len_hbm, feat_hbm, out_hbm,
            gcx_v, gcy_v, i0_v, i1_v, w_v, len_v, fbuf, acc_v,
            sem_in, sem_f, sem_out):
        wid = lax.axis_index("s") * _NC + lax.axis_index("c")
        b = base + wid
        f_cps = [pltpu.async_copy(feat_hbm.at[b, pl.ds(0, _TCH)], fbuf.at[0],
                                  sem_f)]
        cps = [pltpu.async_copy(gc_hbm.at[0], gcx_v, sem_in),
               pltpu.async_copy(gc_hbm.at[1], gcy_v, sem_in),
               pltpu.async_copy(idx0_hbm.at[b], i0_v, sem_in),
               pltpu.async_copy(idx1_hbm.at[b], i1_v, sem_in),
               pltpu.async_copy(len_hbm, len_v, sem_in)]
        for c in cps:
            c.wait()
        len16 = plsc.load_gather(len_v, [jnp.full((_LANES,), b, jnp.int32)])

        def wbody(c, s_acc):
            off = c * _LANES
            v0 = i0_v[pl.ds(off, _LANES)]
            v1 = i1_v[pl.ds(off, _LANES)]
            f = plsc.load_gather(gcx_v, [v0]) + plsc.load_gather(gcy_v, [v1])
            pos = lax.iota(jnp.int32, _LANES) + off
            wc = jnp.where(pos < len16, 1.0 / f, 0.0)
            w_v[pl.ds(off, _LANES)] = wc
            return s_acc + wc

        s_acc = lax.fori_loop(0, wchunks, wbody,
                              jnp.zeros((_LANES,), jnp.float32), unroll=2)
        rcp16 = 1.0 / jnp.full((_LANES,), jnp.sum(s_acc), jnp.float32)
        zero = jnp.zeros((_LANES,), jnp.float32)
        for k in range(dch):
            acc_v[pl.ds(k * _LANES, _LANES)] = zero

        def chunk_body(t, carry):
            pltpu.make_async_copy(feat_hbm.at[b, pl.ds(0, _TCH)],
                                  fbuf.at[0], sem_f).wait()
            buf = lax.rem(t, 2)

            @pl.when(t + 1 < nt)
            def _():
                pltpu.async_copy(feat_hbm.at[b, pl.ds((t + 1) * _TCH, _TCH)],
                                 fbuf.at[lax.rem(t + 1, 2)], sem_f)

            tb = 8

            def block_body(blk, carry2):
                i0 = blk * tb
                tok = t * _TCH + i0
                ws = [plsc.load_gather(
                    w_v, [jnp.full((_LANES,), tok + j, jnp.int32)])
                    for j in range(tb)]

                for k in range(dch):
                    off = k * _LANES
                    val = fbuf[buf, i0, pl.ds(off, _LANES)] * ws[0]
                    for j in range(1, tb):
                        val = val + fbuf[buf, i0 + j,
                                         pl.ds(off, _LANES)] * ws[j]
                    plsc.addupdate(acc_v.at[pl.ds(off, _LANES)], val)
                return carry2

            lax.fori_loop(0, _TCH // tb, block_body, 0)
            return carry

        lax.fori_loop(0, nt, chunk_body, 0)
        for k in range(dch):
            off = k * _LANES
            acc_v[pl.ds(off, _LANES)] = acc_v[pl.ds(off, _LANES)] * rcp16
        pltpu.async_copy(acc_v, out_hbm.at[wid], sem_out).wait()

    return run(idx0, idx1, gc, lengths, feat)


_BB = 2


def _pool_kernel(len_ref, freq_ref, feat_ref, out_ref):
    i = pl.program_id(0)
    bb, _, l = freq_ref.shape
    pos = lax.broadcasted_iota(jnp.int32, (1, l), 1)
    accs = []
    for k in range(bb):
        n = len_ref[i * bb + k]
        w = jnp.where(pos < n, 1.0 / freq_ref[k], 0.0)
        s = jnp.sum(w)
        acc = jnp.dot(w, feat_ref[k], preferred_element_type=jnp.float32)
        accs.append(acc / s)
    out_ref[0] = jnp.concatenate(accs, axis=0)


def _pool(lengths, freq, feat):
    nb, l = freq.shape
    _, _, d = feat.shape
    bb = _BB
    grid_spec = pltpu.PrefetchScalarGridSpec(
        num_scalar_prefetch=1,
        grid=(nb // bb,),
        in_specs=[
            pl.BlockSpec((bb, 1, l), lambda i, *_: (i, 0, 0)),
            pl.BlockSpec((bb, l, d), lambda i, *_: (i, 0, 0)),
        ],
        out_specs=pl.BlockSpec((1, bb, d), lambda i, *_: (i, 0, 0)),
    )
    return pl.pallas_call(
        _pool_kernel,
        grid_spec=grid_spec,
        out_shape=jax.ShapeDtypeStruct((nb // bb, bb, d), jnp.float32),
    )(lengths, freq.reshape(nb, 1, l), feat).reshape(nb, d)


def kernel(input_feature, input_lengths, vq_indices, freqs):
    feat = input_feature[:, -1]
    b = feat.shape[0]
    half = b // 2
    idx0 = vq_indices[:, :, 0]
    idx1 = vq_indices[:, :, 1]
    gc = _compute_gc(freqs)
    freq_a = _sc_gather(idx0[:half], idx1[:half], gc)
    out_b = _sc_pool(idx0, idx1, gc, input_lengths, feat, half)
    out_a = _pool(input_lengths, freq_a, feat)
    return jnp.concatenate([out_a, out_b], axis=0)


# speedup vs baseline: 1.2820x; 1.2820x over previous
import functools

import jax
import jax.numpy as jnp
from jax import lax
from jax.experimental import pallas as pl
from jax.experimental.pallas import tpu as pltpu
from jax.experimental.pallas import tpu_sc as plsc

_NC, _NS, _LANES = 2, 16, 16


def _gc_kernel(freqs_ref, gc_ref):
    f = freqs_ref[...]
    g = f.shape[0]
    ones = jnp.ones((1, g), jnp.float32)
    gcx = lax.dot_general(ones, f, (((1,), (1,)), ((), ())),
                          preferred_element_type=jnp.float32)
    gcy = lax.dot_general(ones, f, (((1,), (0,)), ((), ())),
                          preferred_element_type=jnp.float32)
    gc_ref[...] = jnp.concatenate([gcx, gcy], axis=0)


def _compute_gc(freqs):
    g = freqs.shape[0]
    return pl.pallas_call(
        _gc_kernel,
        out_shape=jax.ShapeDtypeStruct((2, g), jnp.float32),
    )(freqs)


def _sc_mesh():
    return plsc.VectorSubcoreMesh(core_axis_name="c", subcore_axis_name="s",
                                  num_cores=_NC, num_subcores=_NS)


def _sc_gather(idx0, idx1, gc):
    b, l = idx0.shape
    g = gc.shape[1]
    nw = _NC * _NS
    bpw = b // nw
    chunks = l // _LANES

    @functools.partial(
        pl.kernel,
        out_type=jax.ShapeDtypeStruct((b, l), jnp.float32),
        mesh=_sc_mesh(),
        compiler_params=pltpu.CompilerParams(needs_layout_passes=False),
        scratch_types=[
            pltpu.VMEM((g,), jnp.float32),
            pltpu.VMEM((g,), jnp.float32),
            pltpu.VMEM((bpw, l), jnp.int32),
            pltpu.VMEM((bpw, l), jnp.int32),
            pltpu.VMEM((bpw, l), jnp.float32),
            pltpu.SemaphoreType.DMA,
            pltpu.SemaphoreType.DMA,
        ],
    )
    def run(idx0_hbm, idx1_hbm, gc_hbm, out_hbm, gcx_v, gcy_v, i0_v, i1_v,
            f_v, sem_in, sem_out):
        wid = lax.axis_index("s") * _NC + lax.axis_index("c")
        base = wid * bpw
        cps = [pltpu.async_copy(gc_hbm.at[0], gcx_v, sem_in),
               pltpu.async_copy(gc_hbm.at[1], gcy_v, sem_in)]
        for j in range(bpw):
            cps.append(pltpu.async_copy(idx0_hbm.at[base + j], i0_v.at[j], sem_in))
            cps.append(pltpu.async_copy(idx1_hbm.at[base + j], i1_v.at[j], sem_in))
        for c in cps:
            c.wait()
        out_cps = []
        for j in range(bpw):
            def body(c, carry, j=j):
                off = c * _LANES
                v0 = i0_v[j, pl.ds(off, _LANES)]
                v1 = i1_v[j, pl.ds(off, _LANES)]
                fx = plsc.load_gather(gcx_v, [v0])
                fy = plsc.load_gather(gcy_v, [v1])
                f_v[j, pl.ds(off, _LANES)] = fx + fy
                return carry

            lax.fori_loop(0, chunks, body, 0, unroll=4)
            out_cps.append(pltpu.async_copy(f_v.at[j], out_hbm.at[base + j],
                                            sem_out))
        for c in out_cps:
            c.wait()

    return run(idx0, idx1, gc)


_TCH = 64


def _sc_pool(idx0, idx1, gc, lengths, feat, base):
    b_total, l = idx0.shape
    _, _, d = feat.shape
    g = gc.shape[1]
    nw = _NC * _NS
    nb = nw
    nt = l // _TCH
    dch = d // _LANES
    wchunks = l // _LANES

    @functools.partial(
        pl.kernel,
        out_type=jax.ShapeDtypeStruct((nb, d), jnp.float32),
        mesh=_sc_mesh(),
        compiler_params=pltpu.CompilerParams(needs_layout_passes=False),
        scratch_types=[
            pltpu.VMEM((g,), jnp.float32),
            pltpu.VMEM((g,), jnp.float32),
            pltpu.VMEM((l,), jnp.int32),
            pltpu.VMEM((l,), jnp.int32),
            pltpu.VMEM((l,), jnp.float32),
            pltpu.VMEM((b_total,), jnp.int32),
            pltpu.VMEM((2, _TCH, d), jnp.float32),
            pltpu.VMEM((d,), jnp.float32),
            pltpu.SemaphoreType.DMA,
            pltpu.SemaphoreType.DMA,
            pltpu.SemaphoreType.DMA,
        ],
    )
    def run(idx0_hbm, idx1_hbm, gc_hbm, len_hbm, feat_hbm, out_hbm,
            gcx_v, gcy_v, i0_v, i1_v, w_v, len_v, fbuf, acc_v,
            sem_in, sem_f, sem_out):
        wid = lax.axis_index("s") * _NC + lax.axis_index("c")
        b = base + wid
        f_cps = [pltpu.async_copy(feat_hbm.at[b, pl.ds(0, _TCH)], fbuf.at[0],
                                  sem_f)]
        cps = [pltpu.async_copy(gc_hbm.at[0], gcx_v, sem_in),
               pltpu.async_copy(gc_hbm.at[1], gcy_v, sem_in),
               pltpu.async_copy(idx0_hbm.at[b], i0_v, sem_in),
               pltpu.async_copy(idx1_hbm.at[b], i1_v, sem_in),
               pltpu.async_copy(len_hbm, len_v, sem_in)]
        for c in cps:
            c.wait()
        len16 = plsc.load_gather(len_v, [jnp.full((_LANES,), b, jnp.int32)])

        def wbody(c, s_acc):
            off = c * _LANES
            v0 = i0_v[pl.ds(off, _LANES)]
            v1 = i1_v[pl.ds(off, _LANES)]
            f = plsc.load_gather(gcx_v, [v0]) + plsc.load_gather(gcy_v, [v1])
            pos = lax.iota(jnp.int32, _LANES) + off
            wc = jnp.where(pos < len16, 1.0 / f, 0.0)
            w_v[pl.ds(off, _LANES)] = wc
            return s_acc + wc

        s_acc = lax.fori_loop(0, wchunks, wbody,
                              jnp.zeros((_LANES,), jnp.float32), unroll=2)
        rcp16 = 1.0 / jnp.full((_LANES,), jnp.sum(s_acc), jnp.float32)
        zero = jnp.zeros((_LANES,), jnp.float32)
        for k in range(dch):
            acc_v[pl.ds(k * _LANES, _LANES)] = zero

        def chunk_body(t, carry):
            pltpu.make_async_copy(feat_hbm.at[b, pl.ds(0, _TCH)],
                                  fbuf.at[0], sem_f).wait()
            buf = lax.rem(t, 2)

            @pl.when(t + 1 < nt)
            def _():
                pltpu.async_copy(feat_hbm.at[b, pl.ds((t + 1) * _TCH, _TCH)],
                                 fbuf.at[lax.rem(t + 1, 2)], sem_f)

            tb = 8

            def block_body(blk, carry2):
                i0 = blk * tb
                tok = t * _TCH + i0
                ws = [plsc.load_gather(
                    w_v, [jnp.full((_LANES,), tok + j, jnp.int32)])
                    for j in range(tb)]

                def kbody(k):
                    off = k * _LANES
                    val = fbuf[buf, i0, pl.ds(off, _LANES)] * ws[0]
                    for j in range(1, tb):
                        val = val + fbuf[buf, i0 + j,
                                         pl.ds(off, _LANES)] * ws[j]
                    plsc.addupdate(acc_v.at[pl.ds(off, _LANES)], val)

                plsc.parallel_loop(0, dch, unroll=8)(kbody)
                return carry2

            lax.fori_loop(0, _TCH // tb, block_body, 0)
            return carry

        lax.fori_loop(0, nt, chunk_body, 0)
        for k in range(dch):
            off = k * _LANES
            acc_v[pl.ds(off, _LANES)] = acc_v[pl.ds(off, _LANES)] * rcp16
        pltpu.async_copy(acc_v, out_hbm.at[wid], sem_out).wait()

    return run(idx0, idx1, gc, lengths, feat)


_BB = 2


def _pool_kernel(len_ref, freq_ref, feat_ref, out_ref):
    i = pl.program_id(0)
    bb, _, l = freq_ref.shape
    pos = lax.broadcasted_iota(jnp.int32, (1, l), 1)
    accs = []
    for k in range(bb):
        n = len_ref[i * bb + k]
        w = jnp.where(pos < n, 1.0 / freq_ref[k], 0.0)
        s = jnp.sum(w)
        acc = jnp.dot(w, feat_ref[k], preferred_element_type=jnp.float32)
        accs.append(acc / s)
    out_ref[0] = jnp.concatenate(accs, axis=0)


def _pool(lengths, freq, feat):
    nb, l = freq.shape
    _, _, d = feat.shape
    bb = _BB
    grid_spec = pltpu.PrefetchScalarGridSpec(
        num_scalar_prefetch=1,
        grid=(nb // bb,),
        in_specs=[
            pl.BlockSpec((bb, 1, l), lambda i, *_: (i, 0, 0)),
            pl.BlockSpec((bb, l, d), lambda i, *_: (i, 0, 0)),
        ],
        out_specs=pl.BlockSpec((1, bb, d), lambda i, *_: (i, 0, 0)),
    )
    return pl.pallas_call(
        _pool_kernel,
        grid_spec=grid_spec,
        out_shape=jax.ShapeDtypeStruct((nb // bb, bb, d), jnp.float32),
    )(lengths, freq.reshape(nb, 1, l), feat).reshape(nb, d)


def kernel(input_feature, input_lengths, vq_indices, freqs):
    feat = input_feature[:, -1]
    b = feat.shape[0]
    half = b // 2
    idx0 = vq_indices[:, :, 0]
    idx1 = vq_indices[:, :, 1]
    gc = _compute_gc(freqs)
    freq_a = _sc_gather(idx0[:half], idx1[:half], gc)
    out_b = _sc_pool(idx0, idx1, gc, input_lengths, feat, half)
    out_a = _pool(input_lengths, freq_a, feat)
    return jnp.concatenate([out_a, out_b], axis=0)


# speedup vs baseline: 1.3162x; 1.0267x over previous
import functools

import jax
import jax.numpy as jnp
from jax import lax
from jax.experimental import pallas as pl
from jax.experimental.pallas import tpu as pltpu
from jax.experimental.pallas import tpu_sc as plsc

_NC, _NS, _LANES = 2, 16, 16


def _gc_kernel(freqs_ref, gc_ref):
    f = freqs_ref[...]
    g = f.shape[0]
    ones = jnp.ones((1, g), jnp.float32)
    gcx = lax.dot_general(ones, f, (((1,), (1,)), ((), ())),
                          preferred_element_type=jnp.float32)
    gcy = lax.dot_general(ones, f, (((1,), (0,)), ((), ())),
                          preferred_element_type=jnp.float32)
    gc_ref[...] = jnp.concatenate([gcx, gcy], axis=0)


def _compute_gc(freqs):
    g = freqs.shape[0]
    return pl.pallas_call(
        _gc_kernel,
        out_shape=jax.ShapeDtypeStruct((2, g), jnp.float32),
    )(freqs)


def _sc_mesh():
    return plsc.VectorSubcoreMesh(core_axis_name="c", subcore_axis_name="s",
                                  num_cores=_NC, num_subcores=_NS)


def _sc_gather(idx0, idx1, gc):
    b, l = idx0.shape
    g = gc.shape[1]
    nw = _NC * _NS
    bpw = b // nw
    chunks = l // _LANES

    @functools.partial(
        pl.kernel,
        out_type=jax.ShapeDtypeStruct((b, l), jnp.float32),
        mesh=_sc_mesh(),
        compiler_params=pltpu.CompilerParams(needs_layout_passes=False),
        scratch_types=[
            pltpu.VMEM((g,), jnp.float32),
            pltpu.VMEM((g,), jnp.float32),
            pltpu.VMEM((bpw, l), jnp.int32),
            pltpu.VMEM((bpw, l), jnp.int32),
            pltpu.VMEM((bpw, l), jnp.float32),
            pltpu.SemaphoreType.DMA,
            pltpu.SemaphoreType.DMA,
        ],
    )
    def run(idx0_hbm, idx1_hbm, gc_hbm, out_hbm, gcx_v, gcy_v, i0_v, i1_v,
            f_v, sem_in, sem_out):
        wid = lax.axis_index("s") * _NC + lax.axis_index("c")
        base = wid * bpw
        cps = [pltpu.async_copy(gc_hbm.at[0], gcx_v, sem_in),
               pltpu.async_copy(gc_hbm.at[1], gcy_v, sem_in)]
        for j in range(bpw):
            cps.append(pltpu.async_copy(idx0_hbm.at[base + j], i0_v.at[j], sem_in))
            cps.append(pltpu.async_copy(idx1_hbm.at[base + j], i1_v.at[j], sem_in))
        for c in cps:
            c.wait()
        out_cps = []
        for j in range(bpw):
            def body(c, carry, j=j):
                off = c * _LANES
                v0 = i0_v[j, pl.ds(off, _LANES)]
                v1 = i1_v[j, pl.ds(off, _LANES)]
                fx = plsc.load_gather(gcx_v, [v0])
                fy = plsc.load_gather(gcy_v, [v1])
                f_v[j, pl.ds(off, _LANES)] = fx + fy
                return carry

            lax.fori_loop(0, chunks, body, 0, unroll=4)
            out_cps.append(pltpu.async_copy(f_v.at[j], out_hbm.at[base + j],
                                            sem_out))
        for c in out_cps:
            c.wait()

    return run(idx0, idx1, gc)


_TCH = 64


def _sc_pool(idx0, idx1, gc, lengths, feat, base, nb):
    b_total, l = idx0.shape
    _, _, d = feat.shape
    g = gc.shape[1]
    nw = _NC * _NS
    wpb = nw // nb
    dw = d // wpb
    nt = l // _TCH
    dch = dw // _LANES
    wchunks = l // _LANES

    @functools.partial(
        pl.kernel,
        out_type=jax.ShapeDtypeStruct((nb, d), jnp.float32),
        mesh=_sc_mesh(),
        compiler_params=pltpu.CompilerParams(needs_layout_passes=False),
        scratch_types=[
            pltpu.VMEM((g,), jnp.float32),
            pltpu.VMEM((g,), jnp.float32),
            pltpu.VMEM((l,), jnp.int32),
            pltpu.VMEM((l,), jnp.int32),
            pltpu.VMEM((l,), jnp.float32),
            pltpu.VMEM((b_total,), jnp.int32),
            pltpu.VMEM((2, _TCH, dw), jnp.float32),
            pltpu.VMEM((dw,), jnp.float32),
            pltpu.SemaphoreType.DMA,
            pltpu.SemaphoreType.DMA,
            pltpu.SemaphoreType.DMA,
        ],
    )
    def run(idx0_hbm, idx1_hbm, gc_hbm, len_hbm, feat_hbm, out_hbm,
            gcx_v, gcy_v, i0_v, i1_v, w_v, len_v, fbuf, acc_v,
            sem_in, sem_f, sem_out):
        wid = lax.axis_index("s") * _NC + lax.axis_index("c")
        b_local = wid // wpb
        b = base + b_local
        d0 = (wid % wpb) * dw
        f_cps = [pltpu.async_copy(
            feat_hbm.at[b, pl.ds(0, _TCH), pl.ds(d0, dw)], fbuf.at[0],
            sem_f)]
        cps = [pltpu.async_copy(gc_hbm.at[0], gcx_v, sem_in),
               pltpu.async_copy(gc_hbm.at[1], gcy_v, sem_in),
               pltpu.async_copy(idx0_hbm.at[b], i0_v, sem_in),
               pltpu.async_copy(idx1_hbm.at[b], i1_v, sem_in),
               pltpu.async_copy(len_hbm, len_v, sem_in)]
        for c in cps:
            c.wait()
        len16 = plsc.load_gather(len_v, [jnp.full((_LANES,), b, jnp.int32)])

        def wbody(c, s_acc):
            off = c * _LANES
            v0 = i0_v[pl.ds(off, _LANES)]
            v1 = i1_v[pl.ds(off, _LANES)]
            f = plsc.load_gather(gcx_v, [v0]) + plsc.load_gather(gcy_v, [v1])
            pos = lax.iota(jnp.int32, _LANES) + off
            wc = jnp.where(pos < len16, 1.0 / f, 0.0)
            w_v[pl.ds(off, _LANES)] = wc
            return s_acc + wc

        s_acc = lax.fori_loop(0, wchunks, wbody,
                              jnp.zeros((_LANES,), jnp.float32), unroll=2)
        rcp16 = 1.0 / jnp.full((_LANES,), jnp.sum(s_acc), jnp.float32)
        zero = jnp.zeros((_LANES,), jnp.float32)
        for k in range(dch):
            acc_v[pl.ds(k * _LANES, _LANES)] = zero

        def chunk_body(t, carry):
            pltpu.make_async_copy(
                feat_hbm.at[b, pl.ds(0, _TCH), pl.ds(d0, dw)], fbuf.at[0],
                sem_f).wait()
            buf = lax.rem(t, 2)

            @pl.when(t + 1 < nt)
            def _():
                pltpu.async_copy(
                    feat_hbm.at[b, pl.ds((t + 1) * _TCH, _TCH),
                                pl.ds(d0, dw)],
                    fbuf.at[lax.rem(t + 1, 2)], sem_f)

            tb = 8

            def block_body(blk, carry2):
                i0 = blk * tb
                tok = t * _TCH + i0
                ws = [plsc.load_gather(
                    w_v, [jnp.full((_LANES,), tok + j, jnp.int32)])
                    for j in range(tb)]

                def kbody(k):
                    off = k * _LANES
                    val = fbuf[buf, i0, pl.ds(off, _LANES)] * ws[0]
                    for j in range(1, tb):
                        val = val + fbuf[buf, i0 + j,
                                         pl.ds(off, _LANES)] * ws[j]
                    plsc.addupdate(acc_v.at[pl.ds(off, _LANES)], val)

                plsc.parallel_loop(0, dch, unroll=8)(kbody)
                return carry2

            lax.fori_loop(0, _TCH // tb, block_body, 0)
            return carry

        lax.fori_loop(0, nt, chunk_body, 0)
        for k in range(dch):
            off = k * _LANES
            acc_v[pl.ds(off, _LANES)] = acc_v[pl.ds(off, _LANES)] * rcp16
        pltpu.async_copy(acc_v, out_hbm.at[b_local, pl.ds(d0, dw)],
                         sem_out).wait()

    return run(idx0, idx1, gc, lengths, feat)


_BB = 2


def _pool_kernel(len_ref, freq_ref, feat_ref, out_ref):
    i = pl.program_id(0)
    bb, _, l = freq_ref.shape
    pos = lax.broadcasted_iota(jnp.int32, (1, l), 1)
    accs = []
    for k in range(bb):
        n = len_ref[i * bb + k]
        w = jnp.where(pos < n, 1.0 / freq_ref[k], 0.0)
        s = jnp.sum(w)
        acc = jnp.dot(w, feat_ref[k], preferred_element_type=jnp.float32)
        accs.append(acc / s)
    out_ref[0] = jnp.concatenate(accs, axis=0)


def _pool(lengths, freq, feat):
    nb, l = freq.shape
    _, _, d = feat.shape
    bb = _BB
    grid_spec = pltpu.PrefetchScalarGridSpec(
        num_scalar_prefetch=1,
        grid=(nb // bb,),
        in_specs=[
            pl.BlockSpec((bb, 1, l), lambda i, *_: (i, 0, 0)),
            pl.BlockSpec((bb, l, d), lambda i, *_: (i, 0, 0)),
        ],
        out_specs=pl.BlockSpec((1, bb, d), lambda i, *_: (i, 0, 0)),
    )
    return pl.pallas_call(
        _pool_kernel,
        grid_spec=grid_spec,
        out_shape=jax.ShapeDtypeStruct((nb // bb, bb, d), jnp.float32),
    )(lengths, freq.reshape(nb, 1, l), feat).reshape(nb, d)


def kernel(input_feature, input_lengths, vq_indices, freqs):
    feat = input_feature[:, -1]
    b = feat.shape[0]
    nsc = 16
    ntc = b - nsc
    idx0 = vq_indices[:, :, 0]
    idx1 = vq_indices[:, :, 1]
    gc = _compute_gc(freqs)
    freq = _sc_gather(idx0, idx1, gc)
    out_b = _sc_pool(idx0, idx1, gc, input_lengths, feat, ntc, nsc)
    out_a = _pool(input_lengths, freq[:ntc], feat)
    return jnp.concatenate([out_a, out_b], axis=0)


# speedup vs baseline: 1.3532x; 1.0281x over previous
import functools

import jax
import jax.numpy as jnp
from jax import lax
from jax.experimental import pallas as pl
from jax.experimental.pallas import tpu as pltpu
from jax.experimental.pallas import tpu_sc as plsc

_NC, _NS, _LANES = 2, 16, 16


def _gc_kernel(freqs_ref, gc_ref):
    f = freqs_ref[...]
    g = f.shape[0]
    ones = jnp.ones((1, g), jnp.float32)
    gcx = lax.dot_general(ones, f, (((1,), (1,)), ((), ())),
                          preferred_element_type=jnp.float32)
    gcy = lax.dot_general(ones, f, (((1,), (0,)), ((), ())),
                          preferred_element_type=jnp.float32)
    gc_ref[...] = jnp.concatenate([gcx, gcy], axis=0)


def _compute_gc(freqs):
    g = freqs.shape[0]
    return pl.pallas_call(
        _gc_kernel,
        out_shape=jax.ShapeDtypeStruct((2, g), jnp.float32),
    )(freqs)


def _sc_gather(idx0, idx1, gc):
    b, l = idx0.shape
    g = gc.shape[1]
    nw = _NC * _NS
    bpw = b // nw
    chunks = l // _LANES
    mesh = plsc.VectorSubcoreMesh(core_axis_name="c", subcore_axis_name="s",
                                  num_cores=_NC, num_subcores=_NS)

    @functools.partial(
        pl.kernel,
        out_type=jax.ShapeDtypeStruct((b, l), jnp.float32),
        mesh=mesh,
        compiler_params=pltpu.CompilerParams(needs_layout_passes=False),
        scratch_types=[
            pltpu.VMEM((g,), jnp.float32),
            pltpu.VMEM((g,), jnp.float32),
            pltpu.VMEM((bpw, l), jnp.int32),
            pltpu.VMEM((bpw, l), jnp.int32),
            pltpu.VMEM((bpw, l), jnp.float32),
            pltpu.SemaphoreType.DMA,
            pltpu.SemaphoreType.DMA,
        ],
    )
    def run(idx0_hbm, idx1_hbm, gc_hbm, out_hbm, gcx_v, gcy_v, i0_v, i1_v,
            f_v, sem_in, sem_out):
        wid = lax.axis_index("s") * _NC + lax.axis_index("c")
        base = wid * bpw
        cps = [pltpu.async_copy(gc_hbm.at[0], gcx_v, sem_in),
               pltpu.async_copy(gc_hbm.at[1], gcy_v, sem_in)]
        for j in range(bpw):
            cps.append(pltpu.async_copy(idx0_hbm.at[base + j], i0_v.at[j], sem_in))
            cps.append(pltpu.async_copy(idx1_hbm.at[base + j], i1_v.at[j], sem_in))
        for c in cps:
            c.wait()
        out_cps = []
        for j in range(bpw):
            def body(c, carry, j=j):
                off = c * _LANES
                v0 = i0_v[j, pl.ds(off, _LANES)]
                v1 = i1_v[j, pl.ds(off, _LANES)]
                fx = plsc.load_gather(gcx_v, [v0])
                fy = plsc.load_gather(gcy_v, [v1])
                f_v[j, pl.ds(off, _LANES)] = fx + fy
                return carry

            lax.fori_loop(0, chunks, body, 0, unroll=4)
            out_cps.append(pltpu.async_copy(f_v.at[j], out_hbm.at[base + j],
                                            sem_out))
        for c in out_cps:
            c.wait()

    return run(idx0, idx1, gc)


_BB = 2


def _pool_kernel(len_ref, freq_ref, feat_ref, out_ref):
    i = pl.program_id(0)
    bb, _, l = freq_ref.shape
    pos = lax.broadcasted_iota(jnp.int32, (1, l), 1)
    accs = []
    for k in range(bb):
        n = len_ref[i * bb + k]
        w = jnp.where(pos < n, 1.0 / freq_ref[k], 0.0)
        s = jnp.sum(w)
        acc = jnp.dot(w, feat_ref[k], preferred_element_type=jnp.float32)
        accs.append(acc / s)
    out_ref[0] = jnp.concatenate(accs, axis=0)


def _pool(lengths, freq, feat):
    b, l, d = feat.shape
    bb = _BB
    grid_spec = pltpu.PrefetchScalarGridSpec(
        num_scalar_prefetch=1,
        grid=(b // bb,),
        in_specs=[
            pl.BlockSpec((bb, 1, l), lambda i, *_: (i, 0, 0)),
            pl.BlockSpec((bb, l, d), lambda i, *_: (i, 0, 0)),
        ],
        out_specs=pl.BlockSpec((1, bb, d), lambda i, *_: (i, 0, 0)),
    )
    return pl.pallas_call(
        _pool_kernel,
        grid_spec=grid_spec,
        out_shape=jax.ShapeDtypeStruct((b // bb, bb, d), jnp.float32),
    )(lengths, freq.reshape(b, 1, l), feat).reshape(b, d)


def kernel(input_feature, input_lengths, vq_indices, freqs):
    feat = input_feature[:, -1]
    idx0 = vq_indices[:, :, 0]
    idx1 = vq_indices[:, :, 1]
    gc = _compute_gc(freqs)
    freq = _sc_gather(idx0, idx1, gc)
    return _pool(input_lengths, freq, feat)
